# baseline (speedup 1.0000x reference)
import functools

import jax
import jax.numpy as jnp
import numpy as np
from jax import lax
from jax.experimental import pallas as pl
from jax.experimental.pallas import tpu as pltpu
from jax.experimental.pallas import tpu_sc as plsc

B = 1024
NUM_FEATURES = 768
INNER = 4096
OUT = 128
NEG = 65536
TEMP = 0.2
M0 = 0.99
K_TOTAL = int(1281167 / 1024) * 300
K_STEP = 0
M_EFF = float(1.0 - (1.0 - M0) * (np.cos(np.pi * K_STEP / K_TOTAL) + 1.0) / 2.0)

_F32 = jnp.float32
_BF16 = jnp.bfloat16


_TILE1 = 512
_NSTEP1 = INNER // _TILE1


def _bn_relu(h, g, be):
    h3 = h.reshape(2, B, h.shape[-1])
    mu = jnp.mean(h3, axis=1, keepdims=True)
    var = jnp.mean((h3 - mu) ** 2, axis=1, keepdims=True)
    g3 = g.reshape(1, 1, -1)
    be3 = be.reshape(1, 1, -1)
    out = (h3 - mu) / jnp.sqrt(var + 1e-5) * g3 + be3
    return jnp.maximum(out, 0.0).reshape(2 * B, h.shape[-1])


def _dot_bf16(a, b):
    return jnp.dot(a.astype(_BF16), b.astype(_BF16),
                   preferred_element_type=_F32)


def _pk_body(im_ref, wp1_ref, wk1_ref, bp1_ref, bk1_ref, gp1_ref, gk1_ref,
             bep1_ref, bek1_ref, wp2_ref, wk2_ref, bp2_ref, bk2_ref,
             proj_ref, projk_ref, k1t_ref, k2t_ref, accp_ref, acck_ref):
    j = pl.program_id(0)
    m = M_EFF
    im = im_ref[...].astype(_BF16)

    w1p = wp1_ref[...]
    w1k = m * wk1_ref[...] + (1.0 - m) * w1p
    hp = _dot_bf16(im, w1p) + bp1_ref[...]
    hk = _dot_bf16(im, w1k) + (m * bk1_ref[...] + (1.0 - m) * bp1_ref[...])

    relu_p = _bn_relu(hp, gp1_ref[...], bep1_ref[...])
    gk = m * gk1_ref[...] + (1.0 - m) * gp1_ref[...]
    bek = m * bek1_ref[...] + (1.0 - m) * bep1_ref[...]
    relu_k = _bn_relu(hk, gk, bek)

    w2p = wp2_ref[...]
    w2k = m * wk2_ref[...] + (1.0 - m) * w2p
    pp = _dot_bf16(relu_p, w2p)
    pk = _dot_bf16(relu_k, w2k)

    @pl.when(j == 0)
    def _init():
        accp_ref[...] = pp
        acck_ref[...] = pk

    @pl.when(j > 0)
    def _acc():
        accp_ref[...] += pp
        acck_ref[...] += pk

    @pl.when(j == _NSTEP1 - 1)
    def _fini():
        proj_ref[...] = accp_ref[...] + bp2_ref[...]
        pkf = acck_ref[...] + (m * bk2_ref[...] + (1.0 - m) * bp2_ref[...])
        nrm = jnp.sqrt(jnp.sum(pkf * pkf, axis=1, keepdims=True))
        pkn = pkf / nrm
        projk_ref[...] = pkn
        k1t_ref[...] = pkn[:B].T
        k2t_ref[...] = pkn[B:].T


def _proj_key(im, Wp1, Wk1, bp1, bk1, gp1, gk1, bep1, bek1, Wp2, Wk2,
              bp2, bk2):
    vspec = pl.BlockSpec((1, _TILE1), lambda j: (0, j))
    full = lambda shape: pl.BlockSpec(shape, lambda j: (0, 0))
    return pl.pallas_call(
        _pk_body,
        grid=(_NSTEP1,),
        in_specs=[
            full((2 * B, NUM_FEATURES)),
            pl.BlockSpec((NUM_FEATURES, _TILE1), lambda j: (0, j)),
            pl.BlockSpec((NUM_FEATURES, _TILE1), lambda j: (0, j)),
            vspec, vspec, vspec, vspec, vspec, vspec,
            pl.BlockSpec((_TILE1, OUT), lambda j: (j, 0)),
            pl.BlockSpec((_TILE1, OUT), lambda j: (j, 0)),
            full((1, OUT)), full((1, OUT)),
        ],
        out_specs=[
            full((2 * B, OUT)), full((2 * B, OUT)),
            full((OUT, B)), full((OUT, B)),
        ],
        out_shape=[
            jax.ShapeDtypeStruct((2 * B, OUT), _F32),
            jax.ShapeDtypeStruct((2 * B, OUT), _F32),
            jax.ShapeDtypeStruct((OUT, B), _F32),
            jax.ShapeDtypeStruct((OUT, B), _F32),
        ],
        scratch_shapes=[
            pltpu.VMEM((2 * B, OUT), _F32),
            pltpu.VMEM((2 * B, OUT), _F32),
        ],
        compiler_params=pltpu.CompilerParams(
            dimension_semantics=("arbitrary",)),
    )(im, Wp1, Wk1, bp1, bk1, gp1, gk1, bep1, bek1, Wp2, Wk2, bp2, bk2)


def _pred_body(x_ref, w1_ref, b1_ref, g1_ref, be1_ref, w2_ref, b2_ref,
               out_ref, acc_ref):
    j = pl.program_id(0)
    h = _dot_bf16(x_ref[...], w1_ref[...]) + b1_ref[...]
    r = _bn_relu(h, g1_ref[...], be1_ref[...])
    p = _dot_bf16(r, w2_ref[...])

    @pl.when(j == 0)
    def _init():
        acc_ref[...] = p

    @pl.when(j > 0)
    def _acc():
        acc_ref[...] += p

    @pl.when(j == _NSTEP1 - 1)
    def _fini():
        pf = acc_ref[...] + b2_ref[...]
        nrm = jnp.sqrt(jnp.sum(pf * pf, axis=1, keepdims=True))
        out_ref[...] = pf / nrm


def _predictor(x, W1, b1, g1, be1, W2, b2):
    vspec = pl.BlockSpec((1, _TILE1), lambda j: (0, j))
    full = lambda shape: pl.BlockSpec(shape, lambda j: (0, 0))
    return pl.pallas_call(
        _pred_body,
        grid=(_NSTEP1,),
        in_specs=[
            full((2 * B, OUT)),
            pl.BlockSpec((OUT, _TILE1), lambda j: (0, j)),
            vspec, vspec, vspec,
            pl.BlockSpec((_TILE1, OUT), lambda j: (j, 0)),
            full((1, OUT)),
        ],
        out_specs=full((2 * B, OUT)),
        out_shape=jax.ShapeDtypeStruct((2 * B, OUT), _F32),
        scratch_shapes=[pltpu.VMEM((2 * B, OUT), _F32)],
        compiler_params=pltpu.CompilerParams(
            dimension_semantics=("arbitrary",)),
    )(x, W1, b1, g1, be1, W2, b2)


_CTILE = 2048
_NSTEPC = NEG // _CTILE


def _loss_body(q_ref, k_ref, queue_ref, out_ref, acc_ref):
    j = pl.program_id(0)
    logits = _dot_bf16(q_ref[...], queue_ref[...]) * (1.0 / TEMP)
    e = jnp.exp(logits)
    part = jnp.sum(e.reshape(B, _CTILE // 128, 128), axis=1)

    @pl.when(j == 0)
    def _init():
        acc_ref[...] = part

    @pl.when(j > 0)
    def _acc():
        acc_ref[...] += part

    @pl.when(j == _NSTEPC - 1)
    def _fini():
        l_pos = jnp.sum(q_ref[...] * k_ref[...], axis=1,
                        keepdims=True) * (1.0 / TEMP)
        sumneg = jnp.sum(acc_ref[...], axis=1, keepdims=True)
        total = jnp.exp(l_pos) + sumneg
        out_ref[0, 0] = jnp.mean(jnp.log(total) - l_pos)


def _loss(q, k, queue):
    full = lambda shape: pl.BlockSpec(shape, lambda j: (0, 0))
    return pl.pallas_call(
        _loss_body,
        grid=(_NSTEPC,),
        in_specs=[
            full((B, OUT)),
            full((B, OUT)),
            pl.BlockSpec((OUT, _CTILE), lambda j: (0, j)),
        ],
        out_specs=pl.BlockSpec(memory_space=pltpu.SMEM),
        out_shape=jax.ShapeDtypeStruct((1, 1), _F32),
        scratch_shapes=[pltpu.VMEM((B, 128), _F32)],
        compiler_params=pltpu.CompilerParams(
            dimension_semantics=("arbitrary",)),
    )(q, k, queue)


_NW = 32
_RPW = OUT // _NW


def _enqueue_body(q1_ref, q2_ref, k1t_ref, k2t_ref, o1_ref, o2_ref):
    c = lax.axis_index("c")
    s = lax.axis_index("s")
    wid = s * 2 + c
    r0 = wid * _RPW
    rows = pl.ds(r0, _RPW)
    pltpu.sync_copy(k1t_ref.at[rows, :], o1_ref.at[rows, pl.ds(0, B)])
    pltpu.sync_copy(q1_ref.at[rows, pl.ds(B, NEG - B)],
                    o1_ref.at[rows, pl.ds(B, NEG - B)])
    pltpu.sync_copy(k2t_ref.at[rows, :], o2_ref.at[rows, pl.ds(0, B)])
    pltpu.sync_copy(q2_ref.at[rows, pl.ds(B, NEG - B)],
                    o2_ref.at[rows, pl.ds(B, NEG - B)])


def _enqueue(queue1, queue2, k1t, k2t):
    fn = pl.kernel(
        _enqueue_body,
        mesh=plsc.VectorSubcoreMesh(core_axis_name="c", subcore_axis_name="s"),
        out_type=[
            jax.ShapeDtypeStruct((OUT, NEG), _F32),
            jax.ShapeDtypeStruct((OUT, NEG), _F32),
        ],
    )
    return fn(queue1, queue2, k1t, k2t)


def kernel(im_1, im_2, Wp1, bp1, gp1, betp1, Wp2, bp2,
           Wd1, bd1, gd1, betd1, Wd2, bd2,
           Wk1, bk1, gk1, betk1, Wk2, bk2, queue1, queue2):
    im = jnp.concatenate([im_1, im_2], axis=0)
    r1 = lambda v: v.reshape(1, -1)

    proj, projk, k1t, k2t = _proj_key(
        im, Wp1, Wk1, r1(bp1), r1(bk1), r1(gp1), r1(gk1), r1(betp1),
        r1(betk1), Wp2, Wk2, r1(bp2), r1(bk2))

    pred = _predictor(proj, Wd1, r1(bd1), r1(gd1), r1(betd1), Wd2, r1(bd2))

    loss2 = _loss(pred[:B], projk[B:], queue2)
    loss1 = _loss(pred[B:], projk[:B], queue1)
    loss = (loss2[0, 0] + loss1[0, 0]).astype(_F32)

    queue1_new, queue2_new = _enqueue(queue1, queue2, k1t, k2t)
    return loss, queue1_new, queue2_new


# speedup vs baseline: 5.5306x; 5.5306x over previous
import functools

import jax
import jax.numpy as jnp
import numpy as np
from jax import lax
from jax.experimental import pallas as pl
from jax.experimental.pallas import tpu as pltpu
from jax.experimental.pallas import tpu_sc as plsc

B = 1024
NUM_FEATURES = 768
INNER = 4096
OUT = 128
NEG = 65536
TEMP = 0.2
M0 = 0.99
K_TOTAL = int(1281167 / 1024) * 300
K_STEP = 0
M_EFF = float(1.0 - (1.0 - M0) * (np.cos(np.pi * K_STEP / K_TOTAL) + 1.0) / 2.0)

_F32 = jnp.float32
_BF16 = jnp.bfloat16


_TILE1 = 512
_NSTEP1 = INNER // _TILE1


def _bn_relu(h, g, be):
    h3 = h.reshape(2, B, h.shape[-1])
    mu = jnp.mean(h3, axis=1, keepdims=True)
    var = jnp.mean((h3 - mu) ** 2, axis=1, keepdims=True)
    g3 = g.reshape(1, 1, -1)
    be3 = be.reshape(1, 1, -1)
    out = (h3 - mu) / jnp.sqrt(var + 1e-5) * g3 + be3
    return jnp.maximum(out, 0.0).reshape(2 * B, h.shape[-1])


def _dot_bf16(a, b):
    return jnp.dot(a.astype(_BF16), b.astype(_BF16),
                   preferred_element_type=_F32)


def _pk_body(im_ref, wp1_ref, wk1_ref, bp1_ref, bk1_ref, gp1_ref, gk1_ref,
             bep1_ref, bek1_ref, wp2_ref, wk2_ref, bp2_ref, bk2_ref,
             proj_ref, projk_ref, k1t_ref, k2t_ref, accp_ref, acck_ref):
    j = pl.program_id(0)
    m = M_EFF
    im = im_ref[...].astype(_BF16)

    w1p = wp1_ref[...]
    w1k = m * wk1_ref[...] + (1.0 - m) * w1p
    hp = _dot_bf16(im, w1p) + bp1_ref[...]
    hk = _dot_bf16(im, w1k) + (m * bk1_ref[...] + (1.0 - m) * bp1_ref[...])

    relu_p = _bn_relu(hp, gp1_ref[...], bep1_ref[...])
    gk = m * gk1_ref[...] + (1.0 - m) * gp1_ref[...]
    bek = m * bek1_ref[...] + (1.0 - m) * bep1_ref[...]
    relu_k = _bn_relu(hk, gk, bek)

    w2p = wp2_ref[...]
    w2k = m * wk2_ref[...] + (1.0 - m) * w2p
    pp = _dot_bf16(relu_p, w2p)
    pk = _dot_bf16(relu_k, w2k)

    @pl.when(j == 0)
    def _init():
        accp_ref[...] = pp
        acck_ref[...] = pk

    @pl.when(j > 0)
    def _acc():
        accp_ref[...] += pp
        acck_ref[...] += pk

    @pl.when(j == _NSTEP1 - 1)
    def _fini():
        proj_ref[...] = accp_ref[...] + bp2_ref[...]
        pkf = acck_ref[...] + (m * bk2_ref[...] + (1.0 - m) * bp2_ref[...])
        nrm = jnp.sqrt(jnp.sum(pkf * pkf, axis=1, keepdims=True))
        pkn = pkf / nrm
        projk_ref[...] = pkn
        k1t_ref[...] = pkn[:B].T
        k2t_ref[...] = pkn[B:].T


def _proj_key(im, Wp1, Wk1, bp1, bk1, gp1, gk1, bep1, bek1, Wp2, Wk2,
              bp2, bk2):
    vspec = pl.BlockSpec((1, _TILE1), lambda j: (0, j))
    full = lambda shape: pl.BlockSpec(shape, lambda j: (0, 0))
    return pl.pallas_call(
        _pk_body,
        grid=(_NSTEP1,),
        in_specs=[
            full((2 * B, NUM_FEATURES)),
            pl.BlockSpec((NUM_FEATURES, _TILE1), lambda j: (0, j)),
            pl.BlockSpec((NUM_FEATURES, _TILE1), lambda j: (0, j)),
            vspec, vspec, vspec, vspec, vspec, vspec,
            pl.BlockSpec((_TILE1, OUT), lambda j: (j, 0)),
            pl.BlockSpec((_TILE1, OUT), lambda j: (j, 0)),
            full((1, OUT)), full((1, OUT)),
        ],
        out_specs=[
            full((2 * B, OUT)), full((2 * B, OUT)),
            full((OUT, B)), full((OUT, B)),
        ],
        out_shape=[
            jax.ShapeDtypeStruct((2 * B, OUT), _F32),
            jax.ShapeDtypeStruct((2 * B, OUT), _F32),
            jax.ShapeDtypeStruct((OUT, B), _F32),
            jax.ShapeDtypeStruct((OUT, B), _F32),
        ],
        scratch_shapes=[
            pltpu.VMEM((2 * B, OUT), _F32),
            pltpu.VMEM((2 * B, OUT), _F32),
        ],
        compiler_params=pltpu.CompilerParams(
            dimension_semantics=("arbitrary",)),
    )(im, Wp1, Wk1, bp1, bk1, gp1, gk1, bep1, bek1, Wp2, Wk2, bp2, bk2)


def _pred_body(x_ref, w1_ref, b1_ref, g1_ref, be1_ref, w2_ref, b2_ref,
               out_ref, acc_ref):
    j = pl.program_id(0)
    h = _dot_bf16(x_ref[...], w1_ref[...]) + b1_ref[...]
    r = _bn_relu(h, g1_ref[...], be1_ref[...])
    p = _dot_bf16(r, w2_ref[...])

    @pl.when(j == 0)
    def _init():
        acc_ref[...] = p

    @pl.when(j > 0)
    def _acc():
        acc_ref[...] += p

    @pl.when(j == _NSTEP1 - 1)
    def _fini():
        pf = acc_ref[...] + b2_ref[...]
        nrm = jnp.sqrt(jnp.sum(pf * pf, axis=1, keepdims=True))
        out_ref[...] = pf / nrm


def _predictor(x, W1, b1, g1, be1, W2, b2):
    vspec = pl.BlockSpec((1, _TILE1), lambda j: (0, j))
    full = lambda shape: pl.BlockSpec(shape, lambda j: (0, 0))
    return pl.pallas_call(
        _pred_body,
        grid=(_NSTEP1,),
        in_specs=[
            full((2 * B, OUT)),
            pl.BlockSpec((OUT, _TILE1), lambda j: (0, j)),
            vspec, vspec, vspec,
            pl.BlockSpec((_TILE1, OUT), lambda j: (j, 0)),
            full((1, OUT)),
        ],
        out_specs=full((2 * B, OUT)),
        out_shape=jax.ShapeDtypeStruct((2 * B, OUT), _F32),
        scratch_shapes=[pltpu.VMEM((2 * B, OUT), _F32)],
        compiler_params=pltpu.CompilerParams(
            dimension_semantics=("arbitrary",)),
    )(x, W1, b1, g1, be1, W2, b2)


_CTILE = 2048
_NSTEPC = NEG // _CTILE


def _loss_body(q_ref, k_ref, queue_ref, out_ref, acc_ref):
    j = pl.program_id(0)
    logits = _dot_bf16(q_ref[...], queue_ref[...]) * (1.0 / TEMP)
    e = jnp.exp(logits)
    part = jnp.sum(e.reshape(B, _CTILE // 128, 128), axis=1)

    @pl.when(j == 0)
    def _init():
        acc_ref[...] = part

    @pl.when(j > 0)
    def _acc():
        acc_ref[...] += part

    @pl.when(j == _NSTEPC - 1)
    def _fini():
        l_pos = jnp.sum(q_ref[...] * k_ref[...], axis=1,
                        keepdims=True) * (1.0 / TEMP)
        sumneg = jnp.sum(acc_ref[...], axis=1, keepdims=True)
        total = jnp.exp(l_pos) + sumneg
        out_ref[0, 0] = jnp.mean(jnp.log(total) - l_pos)


def _loss(q, k, queue):
    full = lambda shape: pl.BlockSpec(shape, lambda j: (0, 0))
    return pl.pallas_call(
        _loss_body,
        grid=(_NSTEPC,),
        in_specs=[
            full((B, OUT)),
            full((B, OUT)),
            pl.BlockSpec((OUT, _CTILE), lambda j: (0, j)),
        ],
        out_specs=pl.BlockSpec(memory_space=pltpu.SMEM),
        out_shape=jax.ShapeDtypeStruct((1, 1), _F32),
        scratch_shapes=[pltpu.VMEM((B, 128), _F32)],
        compiler_params=pltpu.CompilerParams(
            dimension_semantics=("arbitrary",)),
    )(q, k, queue)


_NW = 32
_RPW = OUT // _NW
_CHUNK = (NEG - B) // 2


def _enqueue_body(q1_ref, q2_ref, k1t_ref, k2t_ref, o1_ref, o2_ref,
                  tbuf, hbuf, gsem, ssem, hsem):
    c = lax.axis_index("c")
    s = lax.axis_index("s")
    wid = s * 2 + c
    r0 = wid * _RPW
    rows = pl.ds(r0, _RPW)

    for kt_ref, o_ref in ((k1t_ref, o1_ref), (k2t_ref, o2_ref)):
        g = pltpu.make_async_copy(kt_ref.at[rows, :], hbuf, hsem)
        g.start()
        g.wait()
        sc = pltpu.make_async_copy(hbuf, o_ref.at[rows, pl.ds(0, B)], hsem)
        sc.start()
        sc.wait()

    copies = []
    for q_ref, o_ref in ((q1_ref, o1_ref), (q2_ref, o2_ref)):
        for r in range(_RPW):
            for h in range(2):
                col0 = B + h * _CHUNK
                sl = (pl.ds(r0 + r, 1), pl.ds(col0, _CHUNK))
                copies.append((q_ref.at[sl], o_ref.at[sl]))

    scatters = [None] * len(copies)
    for i, (src, dst) in enumerate(copies):
        b = i % 2
        if i >= 2:
            scatters[i - 2].wait()
        g = pltpu.make_async_copy(src, tbuf.at[b], gsem.at[b])
        g.start()
        g.wait()
        sc = pltpu.make_async_copy(tbuf.at[b], dst, ssem.at[b])
        sc.start()
        scatters[i] = sc
    scatters[-2].wait()
    scatters[-1].wait()


def _enqueue(queue1, queue2, k1t, k2t):
    fn = pl.kernel(
        _enqueue_body,
        mesh=plsc.VectorSubcoreMesh(core_axis_name="c", subcore_axis_name="s"),
        out_type=[
            jax.ShapeDtypeStruct((OUT, NEG), _F32),
            jax.ShapeDtypeStruct((OUT, NEG), _F32),
        ],
        scratch_types=[
            pltpu.VMEM((2, 1, _CHUNK), _F32),
            pltpu.VMEM((_RPW, B), _F32),
            pltpu.SemaphoreType.DMA((2,)),
            pltpu.SemaphoreType.DMA((2,)),
            pltpu.SemaphoreType.DMA,
        ],
    )
    return fn(queue1, queue2, k1t, k2t)


def kernel(im_1, im_2, Wp1, bp1, gp1, betp1, Wp2, bp2,
           Wd1, bd1, gd1, betd1, Wd2, bd2,
           Wk1, bk1, gk1, betk1, Wk2, bk2, queue1, queue2):
    im = jnp.concatenate([im_1, im_2], axis=0)
    r1 = lambda v: v.reshape(1, -1)

    proj, projk, k1t, k2t = _proj_key(
        im, Wp1, Wk1, r1(bp1), r1(bk1), r1(gp1), r1(gk1), r1(betp1),
        r1(betk1), Wp2, Wk2, r1(bp2), r1(bk2))

    pred = _predictor(proj, Wd1, r1(bd1), r1(gd1), r1(betd1), Wd2, r1(bd2))

    loss2 = _loss(pred[:B], projk[B:], queue2)
    loss1 = _loss(pred[B:], projk[:B], queue1)
    loss = (loss2[0, 0] + loss1[0, 0]).astype(_F32)

    queue1_new, queue2_new = _enqueue(queue1, queue2, k1t, k2t)
    return loss, queue1_new, queue2_new
